# baseline (speedup 1.0000x reference)
import functools

import jax
import jax.numpy as jnp
from jax.experimental import pallas as pl

BLOCK_C = 2048


def _linear_block_kernel(x_ref, w_ref, b_ref, o_ref):
    x = x_ref[...]
    w = w_ref[...]
    acc = jax.lax.dot_general(
        x, w,
        dimension_numbers=(((1,), (1,)), ((), ())),
        preferred_element_type=jnp.float32,
    )
    o_ref[...] = acc + b_ref[...]


@jax.jit
def kernel(x, W, b):
    batch, k = x.shape
    num_classes = W.shape[0]
    b2 = b.reshape(1, num_classes)
    grid = (pl.cdiv(num_classes, BLOCK_C),)
    out = pl.pallas_call(
        _linear_block_kernel,
        grid=grid,
        in_specs=[
            pl.BlockSpec((batch, k), lambda i: (0, 0)),
            pl.BlockSpec((BLOCK_C, k), lambda i: (i, 0)),
            pl.BlockSpec((1, BLOCK_C), lambda i: (0, i)),
        ],
        out_specs=pl.BlockSpec((batch, BLOCK_C), lambda i: (0, i)),
        out_shape=jax.ShapeDtypeStruct((batch, num_classes), jnp.float32),
    )(x, W, b2)
    return out


# speedup vs baseline: 1.0874x; 1.0874x over previous
import jax
import jax.numpy as jnp
from jax.experimental import pallas as pl

BLOCK_B = 32


def _linear_rows_kernel(x_ref, wt_ref, b_ref, o_ref):
    acc = jax.lax.dot_general(
        x_ref[...], wt_ref[...],
        dimension_numbers=(((1,), (0,)), ((), ())),
        preferred_element_type=jnp.float32,
    )
    o_ref[...] = acc + b_ref[...]


@jax.jit
def kernel(x, W, b):
    batch, k = x.shape
    num_classes = W.shape[0]
    wt = W.T
    b2 = b.reshape(1, num_classes)
    grid = (batch // BLOCK_B,)
    out = pl.pallas_call(
        _linear_rows_kernel,
        grid=grid,
        in_specs=[
            pl.BlockSpec((BLOCK_B, k), lambda i: (i, 0)),
            pl.BlockSpec((k, num_classes), lambda i: (0, 0)),
            pl.BlockSpec((1, num_classes), lambda i: (0, 0)),
        ],
        out_specs=pl.BlockSpec((BLOCK_B, num_classes), lambda i: (i, 0)),
        out_shape=jax.ShapeDtypeStruct((batch, num_classes), jnp.float32),
    )(x, wt, b2)
    return out


# speedup vs baseline: 1.0897x; 1.0022x over previous
import jax
import jax.numpy as jnp
from jax.experimental import pallas as pl
from jax.experimental.pallas import tpu as pltpu

BLOCK_B = 32


def _linear_rows_kernel(x_ref, wt_ref, b_ref, o_ref):
    acc = jax.lax.dot_general(
        x_ref[...], wt_ref[...],
        dimension_numbers=(((1,), (0,)), ((), ())),
        preferred_element_type=jnp.float32,
    )
    o_ref[...] = acc + b_ref[...]


@jax.jit
def kernel(x, W, b):
    batch, k = x.shape
    num_classes = W.shape[0]
    wt = W.T
    b2 = b.reshape(1, num_classes)
    grid = (batch // BLOCK_B,)
    out = pl.pallas_call(
        _linear_rows_kernel,
        grid=grid,
        in_specs=[
            pl.BlockSpec((BLOCK_B, k), lambda i: (i, 0)),
            pl.BlockSpec((k, num_classes), lambda i: (0, 0)),
            pl.BlockSpec((1, num_classes), lambda i: (0, 0)),
        ],
        out_specs=pl.BlockSpec((BLOCK_B, num_classes), lambda i: (i, 0)),
        out_shape=jax.ShapeDtypeStruct((batch, num_classes), jnp.float32),
        compiler_params=pltpu.CompilerParams(
            dimension_semantics=("parallel",),
        ),
    )(x, wt, b2)
    return out
